# Initial kernel scaffold; baseline (speedup 1.0000x reference)
import jax
import jax.numpy as jnp
from jax.experimental import pallas as pl


def kernel(heat, K):
    raise NotImplementedError("write your pallas kernel here")


# speedup vs baseline: 1.0543x; 1.0543x over previous
import jax
import jax.numpy as jnp
from jax.experimental import pallas as pl
from jax.experimental.pallas import tpu as pltpu

_H = 512
_W = 512
_K = 500
_NEG = float("-inf")


def _nms_scores(x):
    ninf_row = jnp.full((1, _W), _NEG, dtype=x.dtype)
    up = jnp.concatenate([x[1:, :], ninf_row], axis=0)
    dn = jnp.concatenate([ninf_row, x[:-1, :]], axis=0)
    m = jnp.maximum(jnp.maximum(up, dn), x)
    ninf_col = jnp.full((_H, 1), _NEG, dtype=x.dtype)
    lt = jnp.concatenate([m[:, 1:], ninf_col], axis=1)
    rt = jnp.concatenate([ninf_col, m[:, :-1]], axis=1)
    hmax = jnp.maximum(jnp.maximum(lt, rt), m)
    return jnp.where(hmax == x, x, 0.0)


def _class_topk_kernel(heat_ref, vals_ref, inds_ref, scores_ref, rowmax_ref):
    x = heat_ref[0, 0, :, :]
    scores_ref[:, :] = _nms_scores(x)
    vals_ref[0, 0, :] = jnp.full((512,), _NEG, dtype=jnp.float32)
    inds_ref[0, 0, :] = jnp.zeros((512,), dtype=jnp.int32)

    rowmax_ref[0, :] = jnp.max(scores_ref[:, :], axis=1)
    ciota = jax.lax.broadcasted_iota(jnp.int32, (1, _W), 1)

    def body(k, carry):
        rm = rowmax_ref[0:1, :]
        m = jnp.max(rm)
        j = jnp.min(jnp.where(rm == m, ciota, _H))
        row = scores_ref[pl.ds(j, 1), :]
        v = jnp.max(row)
        c = jnp.min(jnp.where(row == v, ciota, _W))
        slot = ciota == k
        vals_ref[0, 0:1, :] = jnp.where(slot, v, vals_ref[0, 0:1, :])
        inds_ref[0, 0:1, :] = jnp.where(slot, j * _W + c, inds_ref[0, 0:1, :])
        newrow = jnp.where(ciota == c, _NEG, row)
        scores_ref[pl.ds(j, 1), :] = newrow
        rowmax_ref[0:1, :] = jnp.where(ciota == j, jnp.max(newrow), rm)
        return carry

    jax.lax.fori_loop(0, _K, body, 0)


def _merge_kernel(vals_ref, inds_ref, score_ref, ind_ref, cls_ref,
                  ys_ref, xs_ref, cand_ref):
    cand_ref[:, :] = vals_ref[0, :, :]
    riota = jax.lax.broadcasted_iota(jnp.int32, (10, 512), 0)
    ciota = jax.lax.broadcasted_iota(jnp.int32, (10, 512), 1)

    ciota1 = ciota[0:1, :]

    def body(k, carry):
        cand = cand_ref[:, :]
        v = jnp.max(cand)
        sel = cand == v
        flat = jnp.min(jnp.where(sel, riota * 512 + ciota, 10 * 512))
        cls = flat // 512
        rank = flat - cls * 512
        irow = inds_ref[0, pl.ds(cls, 1), :]
        ind = jnp.max(jnp.where(ciota1 == rank, irow, -1))
        y = (ind // _W).astype(jnp.float32)
        x = (ind % _W).astype(jnp.float32)
        slot = ciota1 == k
        score_ref[0, 0:1, :] = jnp.where(slot, v, score_ref[0, 0:1, :])
        ind_ref[0, 0:1, :] = jnp.where(slot, ind, ind_ref[0, 0:1, :])
        cls_ref[0, 0:1, :] = jnp.where(slot, cls, cls_ref[0, 0:1, :])
        ys_ref[0, 0:1, :] = jnp.where(slot, y, ys_ref[0, 0:1, :])
        xs_ref[0, 0:1, :] = jnp.where(slot, x, xs_ref[0, 0:1, :])
        crow = cand_ref[pl.ds(cls, 1), :]
        cand_ref[pl.ds(cls, 1), :] = jnp.where(ciota1 == rank, _NEG, crow)
        return carry

    score_ref[0, 0, :] = jnp.zeros((512,), dtype=jnp.float32)
    ind_ref[0, 0, :] = jnp.zeros((512,), dtype=jnp.int32)
    cls_ref[0, 0, :] = jnp.zeros((512,), dtype=jnp.int32)
    ys_ref[0, 0, :] = jnp.zeros((512,), dtype=jnp.float32)
    xs_ref[0, 0, :] = jnp.zeros((512,), dtype=jnp.float32)
    jax.lax.fori_loop(0, _K, body, 0)


def kernel(heat, K):
    B, C, H, W = heat.shape
    BC = B * C

    vals, inds = pl.pallas_call(
        _class_topk_kernel,
        grid=(BC,),
        in_specs=[pl.BlockSpec((1, 1, H, W),
                               lambda i: (i // C, i % C, 0, 0))],
        out_specs=[pl.BlockSpec((1, 1, 512), lambda i: (i, 0, 0)),
                   pl.BlockSpec((1, 1, 512), lambda i: (i, 0, 0))],
        out_shape=[jax.ShapeDtypeStruct((BC, 1, 512), jnp.float32),
                   jax.ShapeDtypeStruct((BC, 1, 512), jnp.int32)],
        scratch_shapes=[pltpu.VMEM((H, W), jnp.float32),
                        pltpu.VMEM((1, H), jnp.float32)],
    )(heat)

    vals = vals.reshape(B, C, 512)
    inds = inds.reshape(B, C, 512)

    score, ind, cls, ys, xs = pl.pallas_call(
        _merge_kernel,
        grid=(B,),
        in_specs=[pl.BlockSpec((1, C, 512), lambda i: (i, 0, 0)),
                  pl.BlockSpec((1, C, 512), lambda i: (i, 0, 0))],
        out_specs=[pl.BlockSpec((1, 1, 512), lambda i: (i, 0, 0))] * 5,
        out_shape=[jax.ShapeDtypeStruct((B, 1, 512), jnp.float32),
                   jax.ShapeDtypeStruct((B, 1, 512), jnp.int32),
                   jax.ShapeDtypeStruct((B, 1, 512), jnp.int32),
                   jax.ShapeDtypeStruct((B, 1, 512), jnp.float32),
                   jax.ShapeDtypeStruct((B, 1, 512), jnp.float32)],
        scratch_shapes=[pltpu.VMEM((C, 512), jnp.float32)],
    )(vals, inds)

    return (score[:, 0, :_K], ind[:, 0, :_K], cls[:, 0, :_K],
            ys[:, 0, :_K], xs[:, 0, :_K])


# speedup vs baseline: 1.2369x; 1.1731x over previous
import jax
import jax.numpy as jnp
from jax.experimental import pallas as pl
from jax.experimental.pallas import tpu as pltpu

_H = 512
_W = 512
_K = 500
_G = 8
_B = 4
_C = 10
_NEG = float("-inf")


def _nms_scores(x):
    ninf_row = jnp.full((1, _W), _NEG, dtype=x.dtype)
    up = jnp.concatenate([x[1:, :], ninf_row], axis=0)
    dn = jnp.concatenate([ninf_row, x[:-1, :]], axis=0)
    m = jnp.maximum(jnp.maximum(up, dn), x)
    ninf_col = jnp.full((_H, 1), _NEG, dtype=x.dtype)
    lt = jnp.concatenate([m[:, 1:], ninf_col], axis=1)
    rt = jnp.concatenate([ninf_col, m[:, :-1]], axis=1)
    hmax = jnp.maximum(jnp.maximum(lt, rt), m)
    return jnp.where(hmax == x, x, 0.0)


def _class_topk_kernel(heat_ref, vals_ref, inds_ref, scores_ref, rowmax_ref):
    ciota = jax.lax.broadcasted_iota(jnp.int32, (1, _W), 1)
    for c in range(_G):
        scores_ref[c * _H:(c + 1) * _H, :] = _nms_scores(heat_ref[c, :, :])
        rowmax_ref[c:c + 1, :] = jnp.max(
            scores_ref[c * _H:(c + 1) * _H, :], axis=1).reshape(1, _H)
        vals_ref[c, 0:1, :] = jnp.full((1, 512), _NEG, dtype=jnp.float32)
        inds_ref[c, 0:1, :] = jnp.zeros((1, 512), dtype=jnp.int32)

    def body(k, carry):
        slot = ciota == k
        for c in range(_G):
            rm = rowmax_ref[c:c + 1, :]
            m = jnp.max(rm)
            j = jnp.min(jnp.where(rm == m, ciota, _H))
            row = scores_ref[pl.ds(c * _H + j, 1), :]
            v = jnp.max(row)
            col = jnp.min(jnp.where(row == v, ciota, _W))
            vals_ref[c, 0:1, :] = jnp.where(slot, v, vals_ref[c, 0:1, :])
            inds_ref[c, 0:1, :] = jnp.where(
                slot, j * _W + col, inds_ref[c, 0:1, :])
            newrow = jnp.where(ciota == col, _NEG, row)
            scores_ref[pl.ds(c * _H + j, 1), :] = newrow
            rowmax_ref[c:c + 1, :] = jnp.where(ciota == j, jnp.max(newrow), rm)
        return carry

    jax.lax.fori_loop(0, _K, body, 0)


def _merge_kernel(vals_ref, inds_ref, score_ref, ind_ref, cls_ref,
                  ys_ref, xs_ref, cand_ref):
    riota = jax.lax.broadcasted_iota(jnp.int32, (_C, 512), 0)
    ciota = jax.lax.broadcasted_iota(jnp.int32, (_C, 512), 1)
    ciota1 = ciota[0:1, :]
    for b in range(_B):
        cand_ref[b * _C:(b + 1) * _C, :] = vals_ref[b, :, :]
        score_ref[b, 0:1, :] = jnp.zeros((1, 512), dtype=jnp.float32)
        ind_ref[b, 0:1, :] = jnp.zeros((1, 512), dtype=jnp.int32)
        cls_ref[b, 0:1, :] = jnp.zeros((1, 512), dtype=jnp.int32)
        ys_ref[b, 0:1, :] = jnp.zeros((1, 512), dtype=jnp.float32)
        xs_ref[b, 0:1, :] = jnp.zeros((1, 512), dtype=jnp.float32)

    def body(k, carry):
        slot = ciota1 == k
        for b in range(_B):
            cand = cand_ref[b * _C:(b + 1) * _C, :]
            v = jnp.max(cand)
            flat = jnp.min(jnp.where(cand == v, riota * 512 + ciota,
                                     _C * 512))
            cls = flat // 512
            rank = flat - cls * 512
            irow = inds_ref[b, pl.ds(cls, 1), :]
            ind = jnp.max(jnp.where(ciota1 == rank, irow, -1))
            y = (ind // _W).astype(jnp.float32)
            x = (ind % _W).astype(jnp.float32)
            score_ref[b, 0:1, :] = jnp.where(slot, v, score_ref[b, 0:1, :])
            ind_ref[b, 0:1, :] = jnp.where(slot, ind, ind_ref[b, 0:1, :])
            cls_ref[b, 0:1, :] = jnp.where(slot, cls, cls_ref[b, 0:1, :])
            ys_ref[b, 0:1, :] = jnp.where(slot, y, ys_ref[b, 0:1, :])
            xs_ref[b, 0:1, :] = jnp.where(slot, x, xs_ref[b, 0:1, :])
            crow = cand_ref[pl.ds(b * _C + cls, 1), :]
            cand_ref[pl.ds(b * _C + cls, 1), :] = jnp.where(
                ciota1 == rank, _NEG, crow)
        return carry

    jax.lax.fori_loop(0, _K, body, 0)


def kernel(heat, K):
    B, C, H, W = heat.shape
    BC = B * C
    heat2 = heat.reshape(BC, H, W)

    vals, inds = pl.pallas_call(
        _class_topk_kernel,
        grid=(BC // _G,),
        in_specs=[pl.BlockSpec((_G, H, W), lambda i: (i, 0, 0))],
        out_specs=[pl.BlockSpec((_G, 1, 512), lambda i: (i, 0, 0)),
                   pl.BlockSpec((_G, 1, 512), lambda i: (i, 0, 0))],
        out_shape=[jax.ShapeDtypeStruct((BC, 1, 512), jnp.float32),
                   jax.ShapeDtypeStruct((BC, 1, 512), jnp.int32)],
        scratch_shapes=[pltpu.VMEM((_G * H, W), jnp.float32),
                        pltpu.VMEM((_G, H), jnp.float32)],
    )(heat2)

    vals = vals.reshape(B, C, 512)
    inds = inds.reshape(B, C, 512)

    score, ind, cls, ys, xs = pl.pallas_call(
        _merge_kernel,
        grid=(1,),
        in_specs=[pl.BlockSpec((B, C, 512), lambda i: (0, 0, 0)),
                  pl.BlockSpec((B, C, 512), lambda i: (0, 0, 0))],
        out_specs=[pl.BlockSpec((B, 1, 512), lambda i: (0, 0, 0))] * 5,
        out_shape=[jax.ShapeDtypeStruct((B, 1, 512), jnp.float32),
                   jax.ShapeDtypeStruct((B, 1, 512), jnp.int32),
                   jax.ShapeDtypeStruct((B, 1, 512), jnp.int32),
                   jax.ShapeDtypeStruct((B, 1, 512), jnp.float32),
                   jax.ShapeDtypeStruct((B, 1, 512), jnp.float32)],
        scratch_shapes=[pltpu.VMEM((B * _C, 512), jnp.float32)],
    )(vals, inds)

    return (score[:, 0, :_K], ind[:, 0, :_K], cls[:, 0, :_K],
            ys[:, 0, :_K], xs[:, 0, :_K])


# speedup vs baseline: 1.2380x; 1.0009x over previous
import jax
import jax.numpy as jnp
from jax.experimental import pallas as pl
from jax.experimental.pallas import tpu as pltpu

_H = 512
_W = 512
_K = 500
_G = 8
_B = 4
_C = 10
_NEG = float("-inf")


def _nms_scores(x):
    ninf_row = jnp.full((1, _W), _NEG, dtype=x.dtype)
    up = jnp.concatenate([x[1:, :], ninf_row], axis=0)
    dn = jnp.concatenate([ninf_row, x[:-1, :]], axis=0)
    m = jnp.maximum(jnp.maximum(up, dn), x)
    ninf_col = jnp.full((_H, 1), _NEG, dtype=x.dtype)
    lt = jnp.concatenate([m[:, 1:], ninf_col], axis=1)
    rt = jnp.concatenate([ninf_col, m[:, :-1]], axis=1)
    hmax = jnp.maximum(jnp.maximum(lt, rt), m)
    return jnp.where(hmax == x, x, 0.0)


def _class_topk_kernel(heat_ref, vals_ref, inds_ref, *scores_refs):
    ciota = jax.lax.broadcasted_iota(jnp.int32, (1, _W), 1)
    init = []
    for c in range(_G):
        scores_refs[c][:, :] = _nms_scores(heat_ref[c, :, :])
        init.append((
            jnp.max(scores_refs[c][:, :], axis=1).reshape(1, _H),
            jnp.full((1, 512), _NEG, dtype=jnp.float32),
            jnp.zeros((1, 512), dtype=jnp.int32),
        ))

    def body(k, state):
        slot = ciota == k
        out = []
        for c in range(_G):
            rm, vals, inds = state[c]
            m = jnp.max(rm)
            j = jnp.min(jnp.where(rm == m, ciota, _H))
            row = scores_refs[c][pl.ds(j, 1), :]
            v = jnp.max(row)
            col = jnp.min(jnp.where(row == v, ciota, _W))
            vals = jnp.where(slot, v, vals)
            inds = jnp.where(slot, j * _W + col, inds)
            newrow = jnp.where(ciota == col, _NEG, row)
            scores_refs[c][pl.ds(j, 1), :] = newrow
            rm = jnp.where(ciota == j, jnp.max(newrow), rm)
            out.append((rm, vals, inds))
        return out

    final = jax.lax.fori_loop(0, _K, body, init)
    for c in range(_G):
        vals_ref[c, 0:1, :] = final[c][1]
        inds_ref[c, 0:1, :] = final[c][2]


def _merge_kernel(vals_ref, inds_ref, score_ref, ind_ref, cls_ref,
                  ys_ref, xs_ref, *cand_refs):
    riota = jax.lax.broadcasted_iota(jnp.int32, (_C, 512), 0)
    ciota = jax.lax.broadcasted_iota(jnp.int32, (_C, 512), 1)
    ciota1 = ciota[0:1, :]
    zf = jnp.zeros((1, 512), dtype=jnp.float32)
    zi = jnp.zeros((1, 512), dtype=jnp.int32)
    init = []
    for b in range(_B):
        cand_refs[b][:, :] = vals_ref[b, :, :]
        init.append((zf, zi, zi, zf, zf))

    def body(k, state):
        slot = ciota1 == k
        out = []
        for b in range(_B):
            score, ind_a, cls_a, ys_a, xs_a = state[b]
            cand = cand_refs[b][:, :]
            v = jnp.max(cand)
            flat = jnp.min(jnp.where(cand == v, riota * 512 + ciota,
                                     _C * 512))
            cls = flat // 512
            rank = flat - cls * 512
            irow = inds_ref[b, pl.ds(cls, 1), :]
            ind = jnp.max(jnp.where(ciota1 == rank, irow, -1))
            y = (ind // _W).astype(jnp.float32)
            x = (ind % _W).astype(jnp.float32)
            score = jnp.where(slot, v, score)
            ind_a = jnp.where(slot, ind, ind_a)
            cls_a = jnp.where(slot, cls, cls_a)
            ys_a = jnp.where(slot, y, ys_a)
            xs_a = jnp.where(slot, x, xs_a)
            crow = cand_refs[b][pl.ds(cls, 1), :]
            cand_refs[b][pl.ds(cls, 1), :] = jnp.where(
                ciota1 == rank, _NEG, crow)
            out.append((score, ind_a, cls_a, ys_a, xs_a))
        return out

    final = jax.lax.fori_loop(0, _K, body, init)
    for b in range(_B):
        score_ref[b, 0:1, :] = final[b][0]
        ind_ref[b, 0:1, :] = final[b][1]
        cls_ref[b, 0:1, :] = final[b][2]
        ys_ref[b, 0:1, :] = final[b][3]
        xs_ref[b, 0:1, :] = final[b][4]


def kernel(heat, K):
    B, C, H, W = heat.shape
    BC = B * C
    heat2 = heat.reshape(BC, H, W)

    vals, inds = pl.pallas_call(
        _class_topk_kernel,
        grid=(BC // _G,),
        in_specs=[pl.BlockSpec((_G, H, W), lambda i: (i, 0, 0))],
        out_specs=[pl.BlockSpec((_G, 1, 512), lambda i: (i, 0, 0)),
                   pl.BlockSpec((_G, 1, 512), lambda i: (i, 0, 0))],
        out_shape=[jax.ShapeDtypeStruct((BC, 1, 512), jnp.float32),
                   jax.ShapeDtypeStruct((BC, 1, 512), jnp.int32)],
        scratch_shapes=[pltpu.VMEM((H, W), jnp.float32)
                        for _ in range(_G)],
    )(heat2)

    vals = vals.reshape(B, C, 512)
    inds = inds.reshape(B, C, 512)

    score, ind, cls, ys, xs = pl.pallas_call(
        _merge_kernel,
        grid=(1,),
        in_specs=[pl.BlockSpec((B, C, 512), lambda i: (0, 0, 0)),
                  pl.BlockSpec((B, C, 512), lambda i: (0, 0, 0))],
        out_specs=[pl.BlockSpec((B, 1, 512), lambda i: (0, 0, 0))] * 5,
        out_shape=[jax.ShapeDtypeStruct((B, 1, 512), jnp.float32),
                   jax.ShapeDtypeStruct((B, 1, 512), jnp.int32),
                   jax.ShapeDtypeStruct((B, 1, 512), jnp.int32),
                   jax.ShapeDtypeStruct((B, 1, 512), jnp.float32),
                   jax.ShapeDtypeStruct((B, 1, 512), jnp.float32)],
        scratch_shapes=[pltpu.VMEM((_C, 512), jnp.float32)
                        for _ in range(_B)],
    )(vals, inds)

    return (score[:, 0, :_K], ind[:, 0, :_K], cls[:, 0, :_K],
            ys[:, 0, :_K], xs[:, 0, :_K])


# speedup vs baseline: 22.5754x; 18.2359x over previous
import jax
import jax.numpy as jnp
from jax import lax
from jax.experimental import pallas as pl
from jax.experimental.pallas import tpu as pltpu
from jax.experimental.pallas import tpu_sc as plsc

_H = 512
_W = 512
_K = 500
_B = 4
_C = 10
_IMG = _H * _W
_Q = _IMG // 4
_BLK = 256
_NB = _Q // _BLK
_NEG = float("-inf")

_GDN = lax.GatherDimensionNumbers(offset_dims=(), collapsed_slice_dims=(0,),
                                  start_index_map=(0,))


def _lane_gather(v, idx):
    return lax.gather(v, idx.reshape(16, 1), _GDN, (1,),
                      mode=lax.GatherScatterMode.PROMISE_IN_BOUNDS)


def _iota():
    return lax.iota(jnp.int32, 16)


def _vmax16(v):
    iota = _iota()
    for s in (1, 2, 4, 8):
        v = jnp.maximum(v, _lane_gather(v, iota ^ s))
    return v


def _vsum16(v):
    iota = _iota()
    for s in (1, 2, 4, 8):
        v = v + _lane_gather(v, iota ^ s)
    return v


def _vmin16(v):
    iota = _iota()
    for s in (1, 2, 4, 8):
        v = jnp.minimum(v, _lane_gather(v, iota ^ s))
    return v


def _nms_scores(x):
    ninf_row = jnp.full((1, _W), _NEG, dtype=x.dtype)
    up = jnp.concatenate([x[1:, :], ninf_row], axis=0)
    dn = jnp.concatenate([ninf_row, x[:-1, :]], axis=0)
    m = jnp.maximum(jnp.maximum(up, dn), x)
    ninf_col = jnp.full((_H, 1), _NEG, dtype=x.dtype)
    lt = jnp.concatenate([m[:, 1:], ninf_col], axis=1)
    rt = jnp.concatenate([ninf_col, m[:, :-1]], axis=1)
    hmax = jnp.maximum(jnp.maximum(lt, rt), m)
    return jnp.where(hmax == x, x, 0.0)


def _nms_kernel(heat_ref, out_ref):
    out_ref[0, :, :] = _nms_scores(heat_ref[0, :, :])


def _spf(s):
    return jnp.zeros((16,), jnp.float32) + s


def _spi(s):
    return jnp.zeros((16,), jnp.int32) + s


def _scal(v):
    iota = _iota()
    return ((v + iota) - iota)[0]


def _fetch(ref, addr):
    ac = (addr // 16) * 16
    chunk = ref[pl.ds(ac, 16)]
    return _lane_gather(chunk, _spi(addr - ac))


def _blend(ref, k, val):
    kc = (k // 16) * 16
    cur = ref[pl.ds(kc, 16)]
    ref[pl.ds(kc, 16)] = jnp.where(_iota() == _spi(k - kc), val, cur)


def _sc_kernel(scores, qvals, qinds, svals, sinds,
               score_o, ind_o, cls_o, ys_o, xs_o,
               qbuf, bmax, outv, outi, mvals, minds, ivals, iinds,
               oscore, oind, ocls, oys, oxs):
    core = lax.axis_index("c")
    sub = lax.axis_index("s")
    iota = _iota()
    zeroi = iota * 0
    negf = zeroi.astype(jnp.float32) + _NEG

    def task(t, _):
        q = sub + 16 * t
        img = core * 20 + q // 4
        quar = q % 4
        pltpu.sync_copy(scores.at[pl.ds(img * _IMG + quar * _Q, _Q)], qbuf)

        def bg(g, _):
            def bb(b, acc):
                def bf(j, m):
                    return jnp.maximum(
                        m, qbuf[pl.ds((g * 16 + b) * _BLK + j * 16, 16)])
                m = _vmax16(lax.fori_loop(0, _BLK // 16, bf, negf))
                return jnp.where(iota == _spi(b), m, acc)
            bmax[pl.ds(g * 16, 16)] = lax.fori_loop(0, 16, bb, negf)
            return 0
        lax.fori_loop(0, _NB // 16, bg, 0)

        def oi(i, _):
            outv[pl.ds(i * 16, 16)] = negf
            outi[pl.ds(i * 16, 16)] = zeroi
            return 0
        lax.fori_loop(0, 512 // 16, oi, 0)

        def sel(k, _):
            def p1(i, mv):
                return jnp.maximum(mv, bmax[pl.ds(i * 16, 16)])
            m = _vmax16(lax.fori_loop(0, _NB // 16, p1, negf))

            def p2(i, bm):
                c = bmax[pl.ds(i * 16, 16)]
                return jnp.minimum(bm, jnp.where(c == m, i * 16 + iota, _NB))
            blk = _scal(_vmin16(lax.fori_loop(0, _NB // 16, p2,
                                              zeroi + _NB)))
            bbase = blk * _BLK

            def p3(i, em):
                c = qbuf[pl.ds(bbase + i * 16, 16)]
                return jnp.minimum(em, jnp.where(c == m, i * 16 + iota, _BLK))
            loc = _scal(_vmin16(lax.fori_loop(0, _BLK // 16, p3,
                                              zeroi + _BLK)))
            p = bbase + loc
            _blend(outv, k, m)
            _blend(outi, k, _spi(quar * _Q + p))
            pc = (p // 16) * 16
            w = qbuf[pl.ds(pc, 16)]
            qbuf[pl.ds(pc, 16)] = jnp.where(iota == _spi(p - pc), negf, w)

            def rf(j, nm):
                return jnp.maximum(nm, qbuf[pl.ds(bbase + j * 16, 16)])
            nm = _vmax16(lax.fori_loop(0, _BLK // 16, rf, negf))
            _blend(bmax, blk, nm)
            return 0
        lax.fori_loop(0, _K, sel, 0)

        row = (core * 80 + q) * 512
        pltpu.sync_copy(outv, qvals.at[pl.ds(row, 512)])
        pltpu.sync_copy(outi, qinds.at[pl.ds(row, 512)])
        return 0
    lax.fori_loop(0, 5, task, 0)
    plsc.subcore_barrier()

    def merge_quarters(img_local):
        rbase = (core * 80 + img_local * 4) * 512
        pltpu.sync_copy(qvals.at[pl.ds(rbase, 2048)],
                        mvals.at[pl.ds(0, 2048)])
        pltpu.sync_copy(qinds.at[pl.ds(rbase, 2048)],
                        minds.at[pl.ds(0, 2048)])
        heads = negf
        for qq in range(4):
            heads = jnp.where(iota == qq, _spf(mvals[pl.ds(qq * 512, 16)][0]),
                              heads)

        def step(k, st):
            heads, ptrs = st
            m = _vmax16(heads)
            qs = _scal(_vmin16(jnp.where(heads == m, iota, 16)))
            a = qs * 512 + _scal(_vsum16(jnp.where(iota == qs, ptrs, 0)))
            _blend(ivals, k, m)
            _blend(iinds, k, _fetch(minds, a))
            heads = jnp.where(iota == _spi(qs), _fetch(mvals, a + 1), heads)
            ptrs = jnp.where(iota == _spi(qs), ptrs + 1, ptrs)
            return heads, ptrs
        lax.fori_loop(0, _K, step, (heads, zeroi))
        srow = (core * 20 + img_local) * 512
        pltpu.sync_copy(ivals, svals.at[pl.ds(srow, 512)])
        pltpu.sync_copy(iinds, sinds.at[pl.ds(srow, 512)])

    def tail(i, _):
        ivals[pl.ds(i * 16, 16)] = negf
        iinds[pl.ds(i * 16, 16)] = zeroi
        return 0
    lax.fori_loop(0, 512 // 16, tail, 0)
    merge_quarters(sub)

    @pl.when(sub < 4)
    def _():
        merge_quarters(16 + sub)

    plsc.subcore_barrier()

    @pl.when(sub < 2)
    def _():
        bat = core * 2 + sub
        rbase = bat * _C * 512
        pltpu.sync_copy(svals.at[pl.ds(rbase, _C * 512)],
                        mvals.at[pl.ds(0, _C * 512)])
        pltpu.sync_copy(sinds.at[pl.ds(rbase, _C * 512)],
                        minds.at[pl.ds(0, _C * 512)])
        heads = negf
        for cc in range(_C):
            heads = jnp.where(iota == cc, _spf(mvals[pl.ds(cc * 512, 16)][0]),
                              heads)

        def step(k, st):
            heads, ptrs = st
            m = _vmax16(heads)
            cs = _scal(_vmin16(jnp.where(heads == m, iota, 16)))
            a = cs * 512 + _scal(_vsum16(jnp.where(iota == cs, ptrs, 0)))
            ind = _fetch(minds, a)
            _blend(oscore, k, m)
            _blend(oind, k, ind)
            _blend(ocls, k, _spi(cs))
            _blend(oys, k, lax.shift_right_logical(ind, 9).astype(jnp.float32))
            _blend(oxs, k, jnp.bitwise_and(ind, _W - 1).astype(jnp.float32))
            heads = jnp.where(iota == _spi(cs), _fetch(mvals, a + 1), heads)
            ptrs = jnp.where(iota == _spi(cs), ptrs + 1, ptrs)
            return heads, ptrs
        lax.fori_loop(0, _K, step, (heads, zeroi))
        ob = pl.ds(bat * 512, 512)
        pltpu.sync_copy(oscore, score_o.at[ob])
        pltpu.sync_copy(oind, ind_o.at[ob])
        pltpu.sync_copy(ocls, cls_o.at[ob])
        pltpu.sync_copy(oys, ys_o.at[ob])
        pltpu.sync_copy(oxs, xs_o.at[ob])


def kernel(heat, K):
    B, C, H, W = heat.shape
    BC = B * C
    heat2 = heat.reshape(BC, H, W)

    scores = pl.pallas_call(
        _nms_kernel,
        grid=(BC,),
        in_specs=[pl.BlockSpec((1, H, W), lambda i: (i, 0, 0))],
        out_specs=pl.BlockSpec((1, H, W), lambda i: (i, 0, 0)),
        out_shape=jax.ShapeDtypeStruct((BC, H, W), jnp.float32),
    )(heat2).reshape(BC * H * W)

    f32, i32 = jnp.float32, jnp.int32
    mesh = plsc.VectorSubcoreMesh(core_axis_name="c", subcore_axis_name="s")
    outs = pl.kernel(
        _sc_kernel, mesh=mesh,
        out_type=[
            jax.ShapeDtypeStruct((160 * 512,), f32),
            jax.ShapeDtypeStruct((160 * 512,), i32),
            jax.ShapeDtypeStruct((BC * 512,), f32),
            jax.ShapeDtypeStruct((BC * 512,), i32),
            jax.ShapeDtypeStruct((B * 512,), f32),
            jax.ShapeDtypeStruct((B * 512,), i32),
            jax.ShapeDtypeStruct((B * 512,), i32),
            jax.ShapeDtypeStruct((B * 512,), f32),
            jax.ShapeDtypeStruct((B * 512,), f32),
        ],
        scratch_types=[
            pltpu.VMEM((_Q,), f32),
            pltpu.VMEM((_NB,), f32),
            pltpu.VMEM((512,), f32),
            pltpu.VMEM((512,), i32),
            pltpu.VMEM((5120,), f32),
            pltpu.VMEM((5120,), i32),
            pltpu.VMEM((512,), f32),
            pltpu.VMEM((512,), i32),
            pltpu.VMEM((512,), f32),
            pltpu.VMEM((512,), i32),
            pltpu.VMEM((512,), i32),
            pltpu.VMEM((512,), f32),
            pltpu.VMEM((512,), f32),
        ],
    )(scores)
    score, ind, cls, ys, xs = outs[4:]

    return (score.reshape(B, 512)[:, :_K], ind.reshape(B, 512)[:, :_K],
            cls.reshape(B, 512)[:, :_K], ys.reshape(B, 512)[:, :_K],
            xs.reshape(B, 512)[:, :_K])


# speedup vs baseline: 30.1625x; 1.3361x over previous
import jax
import jax.numpy as jnp
from jax import lax
from jax.experimental import pallas as pl
from jax.experimental.pallas import tpu as pltpu
from jax.experimental.pallas import tpu_sc as plsc

_H = 512
_W = 512
_K = 500
_B = 4
_C = 10
_IMG = _H * _W
_Q = _IMG // 4
_BLK = 128
_NB = _Q // _BLK
_NEG = float("-inf")

_GDN = lax.GatherDimensionNumbers(offset_dims=(), collapsed_slice_dims=(0,),
                                  start_index_map=(0,))


def _lane_gather(v, idx):
    return lax.gather(v, idx.reshape(16, 1), _GDN, (1,),
                      mode=lax.GatherScatterMode.PROMISE_IN_BOUNDS)


def _iota():
    return lax.iota(jnp.int32, 16)


def _vmax16(v):
    iota = _iota()
    for s in (1, 2, 4, 8):
        v = jnp.maximum(v, _lane_gather(v, iota ^ s))
    return v


def _vsum16(v):
    iota = _iota()
    for s in (1, 2, 4, 8):
        v = v + _lane_gather(v, iota ^ s)
    return v


def _vmin16(v):
    iota = _iota()
    for s in (1, 2, 4, 8):
        v = jnp.minimum(v, _lane_gather(v, iota ^ s))
    return v


def _nms_scores(x):
    ninf_row = jnp.full((1, _W), _NEG, dtype=x.dtype)
    up = jnp.concatenate([x[1:, :], ninf_row], axis=0)
    dn = jnp.concatenate([ninf_row, x[:-1, :]], axis=0)
    m = jnp.maximum(jnp.maximum(up, dn), x)
    ninf_col = jnp.full((_H, 1), _NEG, dtype=x.dtype)
    lt = jnp.concatenate([m[:, 1:], ninf_col], axis=1)
    rt = jnp.concatenate([ninf_col, m[:, :-1]], axis=1)
    hmax = jnp.maximum(jnp.maximum(lt, rt), m)
    return jnp.where(hmax == x, x, 0.0)


def _nms_kernel(heat_ref, out_ref):
    out_ref[0, :, :] = _nms_scores(heat_ref[0, :, :])


def _spf(s):
    return jnp.zeros((16,), jnp.float32) + s


def _spi(s):
    return jnp.zeros((16,), jnp.int32) + s


def _scal(v):
    iota = _iota()
    return ((v + iota) - iota)[0]


def _fetch(ref, addr):
    ac = (addr // 16) * 16
    chunk = ref[pl.ds(ac, 16)]
    return _lane_gather(chunk, _spi(addr - ac))


def _blend(ref, k, val):
    kc = (k // 16) * 16
    cur = ref[pl.ds(kc, 16)]
    ref[pl.ds(kc, 16)] = jnp.where(_iota() == _spi(k - kc), val, cur)


def _sc_kernel(scores, qvals, qinds, svals, sinds,
               score_o, ind_o, cls_o, ys_o, xs_o,
               qbuf, bmax, outv, outi, mvals, minds, ivals, iinds,
               oscore, oind, ocls, oys, oxs):
    core = lax.axis_index("c")
    sub = lax.axis_index("s")
    iota = _iota()
    zeroi = iota * 0
    negf = zeroi.astype(jnp.float32) + _NEG

    def task(t, _):
        q = sub + 16 * t
        img = core * 20 + q // 4
        quar = q % 4
        pltpu.sync_copy(scores.at[pl.ds(img * _IMG + quar * _Q, _Q)], qbuf)

        def bg(g, _):
            def bb(b, acc):
                def bf(j, m):
                    return jnp.maximum(
                        m, qbuf[pl.ds((g * 16 + b) * _BLK + j * 16, 16)])
                m = _vmax16(lax.fori_loop(0, _BLK // 16, bf, negf))
                return jnp.where(iota == _spi(b), m, acc)
            bmax[pl.ds(g * 16, 16)] = lax.fori_loop(0, 16, bb, negf)
            return 0
        lax.fori_loop(0, _NB // 16, bg, 0)

        def oi(i, _):
            outv[pl.ds(i * 16, 16)] = negf
            outi[pl.ds(i * 16, 16)] = zeroi
            return 0
        lax.fori_loop(0, 512 // 16, oi, 0)

        def sel(k, smax):
            m = _vmax16(smax)
            g = _scal(_vmin16(jnp.where(smax == m, iota, 16)))
            c0 = bmax[pl.ds(g * 32, 16)]
            c1 = bmax[pl.ds(g * 32 + 16, 16)]
            cand = jnp.minimum(
                jnp.where(c0 == m, g * 32 + iota, _NB),
                jnp.where(c1 == m, g * 32 + 16 + iota, _NB))
            blk = _scal(_vmin16(cand))
            bbase = blk * _BLK

            def p3(i, em):
                c = qbuf[pl.ds(bbase + i * 16, 16)]
                return jnp.minimum(em, jnp.where(c == m, i * 16 + iota, _BLK))
            loc = _scal(_vmin16(lax.fori_loop(0, _BLK // 16, p3,
                                              zeroi + _BLK)))
            p = bbase + loc
            _blend(outv, k, m)
            _blend(outi, k, _spi(quar * _Q + p))
            pc = (p // 16) * 16
            w = qbuf[pl.ds(pc, 16)]
            qbuf[pl.ds(pc, 16)] = jnp.where(iota == _spi(p - pc), negf, w)

            def rf(j, nm):
                return jnp.maximum(nm, qbuf[pl.ds(bbase + j * 16, 16)])
            nm = _vmax16(lax.fori_loop(0, _BLK // 16, rf, negf))
            _blend(bmax, blk, nm)
            d0 = bmax[pl.ds(g * 32, 16)]
            d1 = bmax[pl.ds(g * 32 + 16, 16)]
            gm = _vmax16(jnp.maximum(d0, d1))
            return jnp.where(iota == _spi(g), gm, smax)
        def sinit(gg, acc):
            cm = _vmax16(bmax[pl.ds(gg * 16, 16)])
            return jnp.where(iota == _spi(gg // 2), jnp.maximum(acc, cm),
                             acc)
        smax0 = lax.fori_loop(0, _NB // 16, sinit, negf)
        lax.fori_loop(0, _K, sel, smax0)

        row = (core * 80 + q) * 512
        pltpu.sync_copy(outv, qvals.at[pl.ds(row, 512)])
        pltpu.sync_copy(outi, qinds.at[pl.ds(row, 512)])
        return 0
    lax.fori_loop(0, 5, task, 0)
    plsc.subcore_barrier()

    def merge_quarters(img_local):
        rbase = (core * 80 + img_local * 4) * 512
        pltpu.sync_copy(qvals.at[pl.ds(rbase, 2048)],
                        mvals.at[pl.ds(0, 2048)])
        pltpu.sync_copy(qinds.at[pl.ds(rbase, 2048)],
                        minds.at[pl.ds(0, 2048)])
        heads = negf
        for qq in range(4):
            heads = jnp.where(iota == qq, _spf(mvals[pl.ds(qq * 512, 16)][0]),
                              heads)

        def step(k, st):
            heads, ptrs = st
            m = _vmax16(heads)
            qs = _scal(_vmin16(jnp.where(heads == m, iota, 16)))
            a = qs * 512 + _scal(_vsum16(jnp.where(iota == qs, ptrs, 0)))
            _blend(ivals, k, m)
            _blend(iinds, k, _fetch(minds, a))
            heads = jnp.where(iota == _spi(qs), _fetch(mvals, a + 1), heads)
            ptrs = jnp.where(iota == _spi(qs), ptrs + 1, ptrs)
            return heads, ptrs
        lax.fori_loop(0, _K, step, (heads, zeroi))
        srow = (core * 20 + img_local) * 512
        pltpu.sync_copy(ivals, svals.at[pl.ds(srow, 512)])
        pltpu.sync_copy(iinds, sinds.at[pl.ds(srow, 512)])

    def tail(i, _):
        ivals[pl.ds(i * 16, 16)] = negf
        iinds[pl.ds(i * 16, 16)] = zeroi
        return 0
    lax.fori_loop(0, 512 // 16, tail, 0)
    merge_quarters(sub)

    @pl.when(sub < 4)
    def _():
        merge_quarters(16 + sub)

    plsc.subcore_barrier()

    @pl.when(sub < 2)
    def _():
        bat = core * 2 + sub
        rbase = bat * _C * 512
        pltpu.sync_copy(svals.at[pl.ds(rbase, _C * 512)],
                        mvals.at[pl.ds(0, _C * 512)])
        pltpu.sync_copy(sinds.at[pl.ds(rbase, _C * 512)],
                        minds.at[pl.ds(0, _C * 512)])
        heads = negf
        for cc in range(_C):
            heads = jnp.where(iota == cc, _spf(mvals[pl.ds(cc * 512, 16)][0]),
                              heads)

        def step(k, st):
            heads, ptrs = st
            m = _vmax16(heads)
            cs = _scal(_vmin16(jnp.where(heads == m, iota, 16)))
            a = cs * 512 + _scal(_vsum16(jnp.where(iota == cs, ptrs, 0)))
            ind = _fetch(minds, a)
            _blend(oscore, k, m)
            _blend(oind, k, ind)
            _blend(ocls, k, _spi(cs))
            _blend(oys, k, lax.shift_right_logical(ind, 9).astype(jnp.float32))
            _blend(oxs, k, jnp.bitwise_and(ind, _W - 1).astype(jnp.float32))
            heads = jnp.where(iota == _spi(cs), _fetch(mvals, a + 1), heads)
            ptrs = jnp.where(iota == _spi(cs), ptrs + 1, ptrs)
            return heads, ptrs
        lax.fori_loop(0, _K, step, (heads, zeroi))
        ob = pl.ds(bat * 512, 512)
        pltpu.sync_copy(oscore, score_o.at[ob])
        pltpu.sync_copy(oind, ind_o.at[ob])
        pltpu.sync_copy(ocls, cls_o.at[ob])
        pltpu.sync_copy(oys, ys_o.at[ob])
        pltpu.sync_copy(oxs, xs_o.at[ob])


def kernel(heat, K):
    B, C, H, W = heat.shape
    BC = B * C
    heat2 = heat.reshape(BC, H, W)

    scores = pl.pallas_call(
        _nms_kernel,
        grid=(BC,),
        in_specs=[pl.BlockSpec((1, H, W), lambda i: (i, 0, 0))],
        out_specs=pl.BlockSpec((1, H, W), lambda i: (i, 0, 0)),
        out_shape=jax.ShapeDtypeStruct((BC, H, W), jnp.float32),
    )(heat2).reshape(BC * H * W)

    f32, i32 = jnp.float32, jnp.int32
    mesh = plsc.VectorSubcoreMesh(core_axis_name="c", subcore_axis_name="s")
    outs = pl.kernel(
        _sc_kernel, mesh=mesh,
        out_type=[
            jax.ShapeDtypeStruct((160 * 512,), f32),
            jax.ShapeDtypeStruct((160 * 512,), i32),
            jax.ShapeDtypeStruct((BC * 512,), f32),
            jax.ShapeDtypeStruct((BC * 512,), i32),
            jax.ShapeDtypeStruct((B * 512,), f32),
            jax.ShapeDtypeStruct((B * 512,), i32),
            jax.ShapeDtypeStruct((B * 512,), i32),
            jax.ShapeDtypeStruct((B * 512,), f32),
            jax.ShapeDtypeStruct((B * 512,), f32),
        ],
        scratch_types=[
            pltpu.VMEM((_Q,), f32),
            pltpu.VMEM((_NB,), f32),
            pltpu.VMEM((512,), f32),
            pltpu.VMEM((512,), i32),
            pltpu.VMEM((5120,), f32),
            pltpu.VMEM((5120,), i32),
            pltpu.VMEM((512,), f32),
            pltpu.VMEM((512,), i32),
            pltpu.VMEM((512,), f32),
            pltpu.VMEM((512,), i32),
            pltpu.VMEM((512,), i32),
            pltpu.VMEM((512,), f32),
            pltpu.VMEM((512,), f32),
        ],
    )(scores)
    score, ind, cls, ys, xs = outs[4:]

    return (score.reshape(B, 512)[:, :_K], ind.reshape(B, 512)[:, :_K],
            cls.reshape(B, 512)[:, :_K], ys.reshape(B, 512)[:, :_K],
            xs.reshape(B, 512)[:, :_K])


# speedup vs baseline: 33.7544x; 1.1191x over previous
import jax
import jax.numpy as jnp
from jax import lax
from jax.experimental import pallas as pl
from jax.experimental.pallas import tpu as pltpu
from jax.experimental.pallas import tpu_sc as plsc

_H = 512
_W = 512
_K = 500
_B = 4
_C = 10
_IMG = _H * _W
_Q = _IMG // 4
_BLK = 64
_NB = _Q // _BLK
_NEG = float("-inf")

_GDN = lax.GatherDimensionNumbers(offset_dims=(), collapsed_slice_dims=(0,),
                                  start_index_map=(0,))


def _lane_gather(v, idx):
    return lax.gather(v, idx.reshape(16, 1), _GDN, (1,),
                      mode=lax.GatherScatterMode.PROMISE_IN_BOUNDS)


def _iota():
    return lax.iota(jnp.int32, 16)


def _vmax16(v):
    iota = _iota()
    for s in (1, 2, 4, 8):
        v = jnp.maximum(v, _lane_gather(v, iota ^ s))
    return v


def _vsum16(v):
    iota = _iota()
    for s in (1, 2, 4, 8):
        v = v + _lane_gather(v, iota ^ s)
    return v


def _vmin16(v):
    iota = _iota()
    for s in (1, 2, 4, 8):
        v = jnp.minimum(v, _lane_gather(v, iota ^ s))
    return v


def _nms_scores(x):
    ninf_row = jnp.full((1, _W), _NEG, dtype=x.dtype)
    up = jnp.concatenate([x[1:, :], ninf_row], axis=0)
    dn = jnp.concatenate([ninf_row, x[:-1, :]], axis=0)
    m = jnp.maximum(jnp.maximum(up, dn), x)
    ninf_col = jnp.full((_H, 1), _NEG, dtype=x.dtype)
    lt = jnp.concatenate([m[:, 1:], ninf_col], axis=1)
    rt = jnp.concatenate([ninf_col, m[:, :-1]], axis=1)
    hmax = jnp.maximum(jnp.maximum(lt, rt), m)
    return jnp.where(hmax == x, x, 0.0)


def _nms_kernel(heat_ref, out_ref):
    out_ref[0, :, :] = _nms_scores(heat_ref[0, :, :])


def _spf(s):
    return jnp.zeros((16,), jnp.float32) + s


def _spi(s):
    return jnp.zeros((16,), jnp.int32) + s


def _scal(v):
    iota = _iota()
    return ((v + iota) - iota)[0]


def _fetch(ref, addr):
    ac = (addr // 16) * 16
    chunk = ref[pl.ds(ac, 16)]
    return _lane_gather(chunk, _spi(addr - ac))


def _blend(ref, k, val):
    kc = (k // 16) * 16
    cur = ref[pl.ds(kc, 16)]
    ref[pl.ds(kc, 16)] = jnp.where(_iota() == _spi(k - kc), val, cur)


def _sc_kernel(scores, qvals, qinds, svals, sinds,
               score_o, ind_o, cls_o, ys_o, xs_o,
               qbuf, bmax, outv, outi, mvals, minds, ivals, iinds,
               oscore, oind, ocls, oys, oxs):
    core = lax.axis_index("c")
    sub = lax.axis_index("s")
    iota = _iota()
    zeroi = iota * 0
    negf = zeroi.astype(jnp.float32) + _NEG

    def task(t, _):
        q = sub + 16 * t
        img = core * 20 + q // 4
        quar = q % 4
        pltpu.sync_copy(scores.at[pl.ds(img * _IMG + quar * _Q, _Q)], qbuf)

        def bg(g, _):
            def bb(b, acc):
                def bf(j, m):
                    return jnp.maximum(
                        m, qbuf[pl.ds((g * 16 + b) * _BLK + j * 16, 16)])
                m = _vmax16(lax.fori_loop(0, _BLK // 16, bf, negf))
                return jnp.where(iota == _spi(b), m, acc)
            bmax[pl.ds(g * 16, 16)] = lax.fori_loop(0, 16, bb, negf)
            return 0
        lax.fori_loop(0, _NB // 16, bg, 0)

        def oi(i, _):
            outv[pl.ds(i * 16, 16)] = negf
            outi[pl.ds(i * 16, 16)] = zeroi
            return 0
        lax.fori_loop(0, 512 // 16, oi, 0)

        def sel(k, smax):
            m = _vmax16(smax)
            g = _scal(_vmin16(jnp.where(smax == m, iota, 16)))
            cand = zeroi + _NB
            for cc in range(4):
                ci = bmax[pl.ds(g * 64 + cc * 16, 16)]
                cand = jnp.minimum(
                    cand, jnp.where(ci == m, g * 64 + cc * 16 + iota, _NB))
            blk = _scal(_vmin16(cand))
            bbase = blk * _BLK

            def p3(i, em):
                c = qbuf[pl.ds(bbase + i * 16, 16)]
                return jnp.minimum(em, jnp.where(c == m, i * 16 + iota, _BLK))
            loc = _scal(_vmin16(lax.fori_loop(0, _BLK // 16, p3,
                                              zeroi + _BLK)))
            p = bbase + loc
            _blend(outv, k, m)
            _blend(outi, k, _spi(quar * _Q + p))
            pc = (p // 16) * 16
            w = qbuf[pl.ds(pc, 16)]
            qbuf[pl.ds(pc, 16)] = jnp.where(iota == _spi(p - pc), negf, w)

            def rf(j, nm):
                return jnp.maximum(nm, qbuf[pl.ds(bbase + j * 16, 16)])
            nm = _vmax16(lax.fori_loop(0, _BLK // 16, rf, negf))
            _blend(bmax, blk, nm)
            gm = negf
            for cc in range(4):
                gm = jnp.maximum(gm, bmax[pl.ds(g * 64 + cc * 16, 16)])
            gm = _vmax16(gm)
            return jnp.where(iota == _spi(g), gm, smax)
        def sinit(gg, acc):
            cm = _vmax16(bmax[pl.ds(gg * 16, 16)])
            return jnp.where(iota == _spi(gg // 4), jnp.maximum(acc, cm),
                             acc)
        smax0 = lax.fori_loop(0, _NB // 16, sinit, negf)
        lax.fori_loop(0, _K, sel, smax0)

        row = (core * 80 + q) * 512
        pltpu.sync_copy(outv, qvals.at[pl.ds(row, 512)])
        pltpu.sync_copy(outi, qinds.at[pl.ds(row, 512)])
        return 0
    lax.fori_loop(0, 5, task, 0)
    plsc.subcore_barrier()

    def merge_quarters(img_local):
        rbase = (core * 80 + img_local * 4) * 512
        pltpu.sync_copy(qvals.at[pl.ds(rbase, 2048)],
                        mvals.at[pl.ds(0, 2048)])
        pltpu.sync_copy(qinds.at[pl.ds(rbase, 2048)],
                        minds.at[pl.ds(0, 2048)])
        heads = negf
        for qq in range(4):
            heads = jnp.where(iota == qq, _spf(mvals[pl.ds(qq * 512, 16)][0]),
                              heads)

        def step(k, st):
            heads, ptrs = st
            m = _vmax16(heads)
            qs = _scal(_vmin16(jnp.where(heads == m, iota, 16)))
            a = qs * 512 + _scal(_vsum16(jnp.where(iota == qs, ptrs, 0)))
            _blend(ivals, k, m)
            _blend(iinds, k, _fetch(minds, a))
            heads = jnp.where(iota == _spi(qs), _fetch(mvals, a + 1), heads)
            ptrs = jnp.where(iota == _spi(qs), ptrs + 1, ptrs)
            return heads, ptrs
        lax.fori_loop(0, _K, step, (heads, zeroi))
        srow = (core * 20 + img_local) * 512
        pltpu.sync_copy(ivals, svals.at[pl.ds(srow, 512)])
        pltpu.sync_copy(iinds, sinds.at[pl.ds(srow, 512)])

    def tail(i, _):
        ivals[pl.ds(i * 16, 16)] = negf
        iinds[pl.ds(i * 16, 16)] = zeroi
        return 0
    lax.fori_loop(0, 512 // 16, tail, 0)
    merge_quarters(sub)

    @pl.when(sub < 4)
    def _():
        merge_quarters(16 + sub)

    plsc.subcore_barrier()

    @pl.when(sub < 2)
    def _():
        bat = core * 2 + sub
        rbase = bat * _C * 512
        pltpu.sync_copy(svals.at[pl.ds(rbase, _C * 512)],
                        mvals.at[pl.ds(0, _C * 512)])
        pltpu.sync_copy(sinds.at[pl.ds(rbase, _C * 512)],
                        minds.at[pl.ds(0, _C * 512)])
        heads = negf
        for cc in range(_C):
            heads = jnp.where(iota == cc, _spf(mvals[pl.ds(cc * 512, 16)][0]),
                              heads)

        def step(k, st):
            heads, ptrs = st
            m = _vmax16(heads)
            cs = _scal(_vmin16(jnp.where(heads == m, iota, 16)))
            a = cs * 512 + _scal(_vsum16(jnp.where(iota == cs, ptrs, 0)))
            ind = _fetch(minds, a)
            _blend(oscore, k, m)
            _blend(oind, k, ind)
            _blend(ocls, k, _spi(cs))
            _blend(oys, k, lax.shift_right_logical(ind, 9).astype(jnp.float32))
            _blend(oxs, k, jnp.bitwise_and(ind, _W - 1).astype(jnp.float32))
            heads = jnp.where(iota == _spi(cs), _fetch(mvals, a + 1), heads)
            ptrs = jnp.where(iota == _spi(cs), ptrs + 1, ptrs)
            return heads, ptrs
        lax.fori_loop(0, _K, step, (heads, zeroi))
        ob = pl.ds(bat * 512, 512)
        pltpu.sync_copy(oscore, score_o.at[ob])
        pltpu.sync_copy(oind, ind_o.at[ob])
        pltpu.sync_copy(ocls, cls_o.at[ob])
        pltpu.sync_copy(oys, ys_o.at[ob])
        pltpu.sync_copy(oxs, xs_o.at[ob])


def kernel(heat, K):
    B, C, H, W = heat.shape
    BC = B * C
    heat2 = heat.reshape(BC, H, W)

    scores = pl.pallas_call(
        _nms_kernel,
        grid=(BC,),
        in_specs=[pl.BlockSpec((1, H, W), lambda i: (i, 0, 0))],
        out_specs=pl.BlockSpec((1, H, W), lambda i: (i, 0, 0)),
        out_shape=jax.ShapeDtypeStruct((BC, H, W), jnp.float32),
    )(heat2).reshape(BC * H * W)

    f32, i32 = jnp.float32, jnp.int32
    mesh = plsc.VectorSubcoreMesh(core_axis_name="c", subcore_axis_name="s")
    outs = pl.kernel(
        _sc_kernel, mesh=mesh,
        out_type=[
            jax.ShapeDtypeStruct((160 * 512,), f32),
            jax.ShapeDtypeStruct((160 * 512,), i32),
            jax.ShapeDtypeStruct((BC * 512,), f32),
            jax.ShapeDtypeStruct((BC * 512,), i32),
            jax.ShapeDtypeStruct((B * 512,), f32),
            jax.ShapeDtypeStruct((B * 512,), i32),
            jax.ShapeDtypeStruct((B * 512,), i32),
            jax.ShapeDtypeStruct((B * 512,), f32),
            jax.ShapeDtypeStruct((B * 512,), f32),
        ],
        scratch_types=[
            pltpu.VMEM((_Q,), f32),
            pltpu.VMEM((_NB,), f32),
            pltpu.VMEM((512,), f32),
            pltpu.VMEM((512,), i32),
            pltpu.VMEM((5120,), f32),
            pltpu.VMEM((5120,), i32),
            pltpu.VMEM((512,), f32),
            pltpu.VMEM((512,), i32),
            pltpu.VMEM((512,), f32),
            pltpu.VMEM((512,), i32),
            pltpu.VMEM((512,), i32),
            pltpu.VMEM((512,), f32),
            pltpu.VMEM((512,), f32),
        ],
    )(scores)
    score, ind, cls, ys, xs = outs[4:]

    return (score.reshape(B, 512)[:, :_K], ind.reshape(B, 512)[:, :_K],
            cls.reshape(B, 512)[:, :_K], ys.reshape(B, 512)[:, :_K],
            xs.reshape(B, 512)[:, :_K])
